# baseline (speedup 1.0000x reference)
import jax
import jax.numpy as jnp
from jax.experimental import pallas as pl
from jax.experimental.pallas import tpu as pltpu

B, C, H, W = 128, 768, 14, 14
E = 64
TOP_K = 8
HW = H * W
C_BLK = 128
NEG = -3.0e38


def _gating_body(x_ref, w_ref, out_ref, acc_ref, seg_ref):
    i = pl.program_id(0)

    @pl.when(i == 0)
    def _init():
        acc_ref[...] = jnp.zeros_like(acc_ref)
        rows = jax.lax.broadcasted_iota(jnp.int32, (C_BLK * HW, C_BLK), 0)
        cols = jax.lax.broadcasted_iota(jnp.int32, (C_BLK * HW, C_BLK), 1)
        seg_ref[...] = jnp.where(rows // HW == cols,
                                 jnp.float32(1.0 / HW), jnp.float32(0.0))

    pooled = jnp.dot(x_ref[...], seg_ref[...],
                     precision=jax.lax.Precision.HIGHEST,
                     preferred_element_type=jnp.float32)
    acc_ref[...] += jnp.dot(pooled, w_ref[...],
                            preferred_element_type=jnp.float32)

    @pl.when(i == pl.num_programs(0) - 1)
    def _finish():
        logits = acc_ref[...]
        cols = jax.lax.broadcasted_iota(jnp.int32, (B, E), 1)
        selected = jnp.zeros((B, E), dtype=jnp.bool_)
        avail = logits
        for _ in range(TOP_K):
            m = jnp.max(avail, axis=1, keepdims=True)
            cand = avail == m
            idx = jnp.min(jnp.where(cand, cols, E), axis=1, keepdims=True)
            first = cand & (cols == idx)
            selected = selected | first
            avail = jnp.where(first, NEG, avail)
        mx = jnp.max(jnp.where(selected, logits, NEG), axis=1, keepdims=True)
        ex = jnp.where(selected, jnp.exp(logits - mx), jnp.float32(0.0))
        out_ref[...] = ex / jnp.sum(ex, axis=1, keepdims=True)


@jax.jit
def kernel(x, W_gate):
    x2 = x.reshape(B, C * HW)
    grid = C // C_BLK
    return pl.pallas_call(
        _gating_body,
        grid=(grid,),
        in_specs=[
            pl.BlockSpec((B, C_BLK * HW), lambda i: (0, i)),
            pl.BlockSpec((C_BLK, E), lambda i: (i, 0)),
        ],
        out_specs=pl.BlockSpec((B, E), lambda i: (0, 0)),
        out_shape=jax.ShapeDtypeStruct((B, E), jnp.float32),
        scratch_shapes=[
            pltpu.VMEM((B, E), jnp.float32),
            pltpu.VMEM((C_BLK * HW, C_BLK), jnp.float32),
        ],
    )(x2, W_gate)


# speedup vs baseline: 10.1402x; 10.1402x over previous
import jax
import jax.numpy as jnp
from jax.experimental import pallas as pl
from jax.experimental.pallas import tpu as pltpu

B, C, H, W = 128, 768, 14, 14
E = 64
TOP_K = 8
HW = H * W
HW_BLK = 14
NEG = -3.0e38


def _gating_body(x_ref, w_ref, out_ref, acc_ref):
    i = pl.program_id(0)

    @pl.when(i == 0)
    def _init():
        acc_ref[...] = jnp.zeros_like(acc_ref)

    acc_ref[...] += jnp.sum(x_ref[...], axis=0)

    @pl.when(i == pl.num_programs(0) - 1)
    def _finish():
        pooled = acc_ref[...] * jnp.float32(1.0 / HW)
        logits = jnp.dot(pooled, w_ref[...],
                         preferred_element_type=jnp.float32)
        cols = jax.lax.broadcasted_iota(jnp.int32, (B, E), 1)
        selected = jnp.zeros((B, E), dtype=jnp.bool_)
        avail = logits
        for _ in range(TOP_K):
            m = jnp.max(avail, axis=1, keepdims=True)
            cand = avail == m
            idx = jnp.min(jnp.where(cand, cols, E), axis=1, keepdims=True)
            first = cand & (cols == idx)
            selected = selected | first
            avail = jnp.where(first, NEG, avail)
        mx = jnp.max(jnp.where(selected, logits, NEG), axis=1, keepdims=True)
        ex = jnp.where(selected, jnp.exp(logits - mx), jnp.float32(0.0))
        out_ref[...] = ex / jnp.sum(ex, axis=1, keepdims=True)


@jax.jit
def kernel(x, W_gate):
    xs = jnp.transpose(x, (2, 3, 0, 1)).reshape(HW, B, C)
    return pl.pallas_call(
        _gating_body,
        grid=(HW // HW_BLK,),
        in_specs=[
            pl.BlockSpec((HW_BLK, B, C), lambda i: (i, 0, 0)),
            pl.BlockSpec((C, E), lambda i: (0, 0)),
        ],
        out_specs=pl.BlockSpec((B, E), lambda i: (0, 0)),
        out_shape=jax.ShapeDtypeStruct((B, E), jnp.float32),
        scratch_shapes=[pltpu.VMEM((B, C), jnp.float32)],
    )(xs, W_gate)
